# baseline (speedup 1.0000x reference)
import functools

import jax
import jax.numpy as jnp
from jax import lax
from jax.experimental import pallas as pl
from jax.experimental.pallas import tpu as pltpu
from jax.experimental.pallas import tpu_sc as plsc

_B, _S, _D = 4, 2048, 768
_NC, _NS = 2, 16
_NW = _NC * _NS
_RPW = (_B * _S) // _NW
_CH = 32
_NCHUNK = _RPW // _CH

_BR = 512
_NBLK = _S // _BR


def _gather_body(tok_idx, table, out, idx_v, buf0, buf1,
                 sem_in0, sem_in1, sem_out0, sem_out1):
    wid = lax.axis_index("s") * _NC + lax.axis_index("c")
    pltpu.sync_copy(tok_idx.at[wid], idx_v)
    g0 = wid * _RPW
    bufs = (buf0, buf1)
    sin = (sem_in0, sem_in1)
    sout = (sem_out0, sem_out1)

    def gin(ci, p):
        return pltpu.make_async_copy(table.at[idx_v.at[ci]], bufs[p], sin[p])

    def gout(ci, p):
        return pltpu.make_async_copy(
            bufs[p], out.at[pl.ds(g0 + ci * _CH, _CH)], sout[p])

    gin(0, 0).start()
    for ci in range(_NCHUNK):
        p = ci % 2
        if ci + 1 < _NCHUNK:
            if ci >= 1:
                gout(ci - 1, 1 - p).wait()
            gin(ci + 1, 1 - p).start()
        gin(ci, p).wait()
        gout(ci, p).start()
    for ci in range(_NCHUNK - 2, _NCHUNK):
        gout(ci, ci % 2).wait()


def _sc_gather(tok_idx, table):
    mesh = plsc.VectorSubcoreMesh(core_axis_name="c", subcore_axis_name="s")
    fn = functools.partial(
        pl.kernel,
        mesh=mesh,
        out_type=jax.ShapeDtypeStruct((_B * _S, _D), jnp.float32),
        scratch_types=[
            pltpu.VMEM((_NCHUNK, _CH), jnp.int32),
            pltpu.VMEM((_CH, _D), jnp.float32),
            pltpu.VMEM((_CH, _D), jnp.float32),
            pltpu.SemaphoreType.DMA,
            pltpu.SemaphoreType.DMA,
            pltpu.SemaphoreType.DMA,
            pltpu.SemaphoreType.DMA,
        ],
    )(_gather_body)
    return fn(tok_idx, table)


def _dense_body(seg_ref, st_ref, gam_ref, bet_ref, g_ref, pos_ref, o_ref):
    x = g_ref[...]
    seg = seg_ref[...]
    pos = pos_ref[...]
    p0 = pos + st_ref[0:1, 0:1, :]
    p1 = pos + st_ref[0:1, 1:2, :]
    x = x + jnp.where(seg == 0, p0, jnp.where(seg == 1, p1, pos))
    inv_d = jnp.float32(1.0 / _D)
    mu = jnp.sum(x, axis=2, keepdims=True) * inv_d
    s2 = jnp.sum(x * x, axis=2, keepdims=True) * inv_d
    var = s2 - mu * mu
    inv = lax.rsqrt(var + jnp.float32(1e-5))
    o_ref[...] = (x - mu) * inv * gam_ref[...] + bet_ref[...]


def _tc_dense(gathered, segments, seg_table, positions, gamma, beta):
    return pl.pallas_call(
        _dense_body,
        grid=(_NBLK,),
        in_specs=[
            pl.BlockSpec((_B, _BR, 1), lambda i: (0, i, 0)),
            pl.BlockSpec((1, 3, _D), lambda i: (0, 0, 0)),
            pl.BlockSpec((1, 1, _D), lambda i: (0, 0, 0)),
            pl.BlockSpec((1, 1, _D), lambda i: (0, 0, 0)),
            pl.BlockSpec((_B, _BR, _D), lambda i: (0, i, 0)),
            pl.BlockSpec((1, _BR, _D), lambda i: (0, i, 0)),
        ],
        out_specs=pl.BlockSpec((_B, _BR, _D), lambda i: (0, i, 0)),
        out_shape=jax.ShapeDtypeStruct((_B, _S, _D), jnp.float32),
        compiler_params=pltpu.CompilerParams(
            dimension_semantics=("arbitrary",)),
    )(segments, seg_table, gamma, beta, gathered, positions)


@jax.jit
def kernel(batched_tokens, batched_segments, tokens_table, segments_table,
           positions, gamma, beta):
    tok_idx = batched_tokens.reshape(_NW, _NCHUNK, _CH)
    gathered = _sc_gather(tok_idx, tokens_table)
    return _tc_dense(gathered.reshape(_B, _S, _D),
                     batched_segments.reshape(_B, _S, 1),
                     segments_table.reshape(1, 3, _D),
                     positions.reshape(1, _S, _D),
                     gamma.reshape(1, 1, _D), beta.reshape(1, 1, _D))


# speedup vs baseline: 1.0055x; 1.0055x over previous
import functools

import jax
import jax.numpy as jnp
from jax import lax
from jax.experimental import pallas as pl
from jax.experimental.pallas import tpu as pltpu
from jax.experimental.pallas import tpu_sc as plsc

_B, _S, _D = 4, 2048, 768
_NC, _NS = 2, 16
_NW = _NC * _NS
_RPW = (_B * _S) // _NW
_CH = 64
_NCHUNK = _RPW // _CH

_BR = 512
_NBLK = _S // _BR


def _gather_body(tok_idx, table, out, idx_v, buf0, buf1,
                 sem_in0, sem_in1, sem_out0, sem_out1):
    wid = lax.axis_index("s") * _NC + lax.axis_index("c")
    pltpu.sync_copy(tok_idx.at[wid], idx_v)
    g0 = wid * _RPW
    bufs = (buf0, buf1)
    sin = (sem_in0, sem_in1)
    sout = (sem_out0, sem_out1)

    def gin(ci, p):
        return pltpu.make_async_copy(table.at[idx_v.at[ci]], bufs[p], sin[p])

    def gout(ci, p):
        return pltpu.make_async_copy(
            bufs[p], out.at[pl.ds(g0 + ci * _CH, _CH)], sout[p])

    gin(0, 0).start()
    for ci in range(_NCHUNK):
        p = ci % 2
        if ci + 1 < _NCHUNK:
            if ci >= 1:
                gout(ci - 1, 1 - p).wait()
            gin(ci + 1, 1 - p).start()
        gin(ci, p).wait()
        gout(ci, p).start()
    for ci in range(_NCHUNK - 2, _NCHUNK):
        gout(ci, ci % 2).wait()


def _sc_gather(tok_idx, table):
    mesh = plsc.VectorSubcoreMesh(core_axis_name="c", subcore_axis_name="s")
    fn = functools.partial(
        pl.kernel,
        mesh=mesh,
        out_type=jax.ShapeDtypeStruct((_B * _S, _D), jnp.float32),
        scratch_types=[
            pltpu.VMEM((_NCHUNK, _CH), jnp.int32),
            pltpu.VMEM((_CH, _D), jnp.float32),
            pltpu.VMEM((_CH, _D), jnp.float32),
            pltpu.SemaphoreType.DMA,
            pltpu.SemaphoreType.DMA,
            pltpu.SemaphoreType.DMA,
            pltpu.SemaphoreType.DMA,
        ],
    )(_gather_body)
    return fn(tok_idx, table)


def _dense_body(seg_ref, st_ref, gam_ref, bet_ref, g_ref, pos_ref, o_ref):
    x = g_ref[...]
    seg = seg_ref[...]
    pos = pos_ref[...]
    p0 = pos + st_ref[0:1, 0:1, :]
    p1 = pos + st_ref[0:1, 1:2, :]
    x = x + jnp.where(seg == 0, p0, jnp.where(seg == 1, p1, pos))
    inv_d = jnp.float32(1.0 / _D)
    mu = jnp.sum(x, axis=2, keepdims=True) * inv_d
    s2 = jnp.sum(x * x, axis=2, keepdims=True) * inv_d
    var = s2 - mu * mu
    inv = lax.rsqrt(var + jnp.float32(1e-5))
    o_ref[...] = (x - mu) * inv * gam_ref[...] + bet_ref[...]


def _tc_dense(gathered, segments, seg_table, positions, gamma, beta):
    return pl.pallas_call(
        _dense_body,
        grid=(_NBLK,),
        in_specs=[
            pl.BlockSpec((_B, _BR, 1), lambda i: (0, i, 0)),
            pl.BlockSpec((1, 3, _D), lambda i: (0, 0, 0)),
            pl.BlockSpec((1, 1, _D), lambda i: (0, 0, 0)),
            pl.BlockSpec((1, 1, _D), lambda i: (0, 0, 0)),
            pl.BlockSpec((_B, _BR, _D), lambda i: (0, i, 0)),
            pl.BlockSpec((1, _BR, _D), lambda i: (0, i, 0)),
        ],
        out_specs=pl.BlockSpec((_B, _BR, _D), lambda i: (0, i, 0)),
        out_shape=jax.ShapeDtypeStruct((_B, _S, _D), jnp.float32),
        compiler_params=pltpu.CompilerParams(
            dimension_semantics=("arbitrary",)),
    )(segments, seg_table, gamma, beta, gathered, positions)


@jax.jit
def kernel(batched_tokens, batched_segments, tokens_table, segments_table,
           positions, gamma, beta):
    tok_idx = batched_tokens.reshape(_NW, _NCHUNK, _CH)
    gathered = _sc_gather(tok_idx, tokens_table)
    return _tc_dense(gathered.reshape(_B, _S, _D),
                     batched_segments.reshape(_B, _S, 1),
                     segments_table.reshape(1, 3, _D),
                     positions.reshape(1, _S, _D),
                     gamma.reshape(1, 1, _D), beta.reshape(1, 1, _D))
